# baseline (speedup 1.0000x reference)
import functools

import jax
import jax.numpy as jnp
from jax import lax
from jax.experimental import pallas as pl
from jax.experimental.pallas import tpu as pltpu
from jax.experimental.pallas import tpu_sc as plsc

_B = 16384
_D = 16
_MU = 3.5
_ROWS_PER_SUPER = 128 // _D
_SUPER = 1000000 // _ROWS_PER_SUPER

_INFO = plsc.get_sparse_core_info()
_NC = _INFO.num_cores
_NS = _INFO.num_subcores
_L = _INFO.num_lanes
_NW = _NC * _NS
_BPW = _B // _NW
_CHUNK = _BPW // 2
_NBLK = _BPW // _L
_CBLK = _CHUNK // _L


def _svd_bias_body(user_idx, item_idx, eu_w, ei_w, ub_w, ib_w, out_hbm,
                   idx_u, idx_i, sidx_u, sidx_i, soff_u, soff_i,
                   rows_u, rows_i, ub_v, ib_v, out_v, s0, s1, s2, s3):
    wid = lax.axis_index("s") * _NC + lax.axis_index("c")
    base = wid * _BPW

    pltpu.sync_copy(user_idx.at[pl.ds(base, _BPW)], idx_u)
    pltpu.sync_copy(item_idx.at[pl.ds(base, _BPW)], idx_i)

    cub = pltpu.async_copy(ub_w.at[idx_u], ub_v, s2)
    cib = pltpu.async_copy(ib_w.at[idx_i], ib_v, s3)

    def derive(k, carry):
        b0 = k * _L
        u = idx_u[pl.ds(b0, _L)]
        i = idx_i[pl.ds(b0, _L)]
        sidx_u[pl.ds(b0, _L)] = u >> 3
        sidx_i[pl.ds(b0, _L)] = i >> 3
        soff_u[pl.ds(b0, _L)] = (u & 7) << 4
        soff_i[pl.ds(b0, _L)] = (i & 7) << 4
        return carry

    lax.fori_loop(0, _NBLK, derive, 0)

    lanes = lax.iota(jnp.int32, _L)

    for chunk in range(2):
        c0 = chunk * _CHUNK
        cu = pltpu.async_copy(eu_w.at[sidx_u.at[pl.ds(c0, _CHUNK)]], rows_u, s0)
        ci = pltpu.async_copy(ei_w.at[sidx_i.at[pl.ds(c0, _CHUNK)]], rows_i, s1)
        cu.wait()
        ci.wait()

        def blk(k, carry):
            b0 = k * _L
            row_ids = b0 + lanes
            off_u = soff_u[pl.ds(c0 + b0, _L)]
            off_i = soff_i[pl.ds(c0 + b0, _L)]
            acc = jnp.zeros((_L,), jnp.float32)
            for c in range(_D):
                vu = plsc.load_gather(rows_u, [row_ids, off_u + c])
                vi = plsc.load_gather(rows_i, [row_ids, off_i + c])
                acc = acc + vu * vi
            out_v[pl.ds(c0 + b0, _L)] = acc
            return carry

        lax.fori_loop(0, _CBLK, blk, 0)

    cub.wait()
    cib.wait()

    def bias_blk(k, carry):
        b0 = k * _L
        out_v[pl.ds(b0, _L)] = (out_v[pl.ds(b0, _L)] + ub_v[pl.ds(b0, _L)]
                                + ib_v[pl.ds(b0, _L)] + _MU)
        return carry

    lax.fori_loop(0, _NBLK, bias_blk, 0)

    pltpu.sync_copy(out_v, out_hbm.at[pl.ds(base, _BPW)])


_svd_bias = functools.partial(
    pl.kernel,
    mesh=plsc.VectorSubcoreMesh(core_axis_name="c", subcore_axis_name="s"),
    compiler_params=pltpu.CompilerParams(needs_layout_passes=False),
    out_type=jax.ShapeDtypeStruct((_B,), jnp.float32),
    scratch_types=[
        pltpu.VMEM((_BPW,), jnp.int32),
        pltpu.VMEM((_BPW,), jnp.int32),
        pltpu.VMEM((_BPW,), jnp.int32),
        pltpu.VMEM((_BPW,), jnp.int32),
        pltpu.VMEM((_BPW,), jnp.int32),
        pltpu.VMEM((_BPW,), jnp.int32),
        pltpu.VMEM((_CHUNK, 128), jnp.float32),
        pltpu.VMEM((_CHUNK, 128), jnp.float32),
        pltpu.VMEM((_BPW,), jnp.float32),
        pltpu.VMEM((_BPW,), jnp.float32),
        pltpu.VMEM((_BPW,), jnp.float32),
        pltpu.SemaphoreType.DMA,
        pltpu.SemaphoreType.DMA,
        pltpu.SemaphoreType.DMA,
        pltpu.SemaphoreType.DMA,
    ],
)(_svd_bias_body)


def kernel(user_idx, item_idx, embed_user_w, embed_item_w, user_bias_w, item_bias_w):
    return _svd_bias(user_idx.astype(jnp.int32), item_idx.astype(jnp.int32),
                     embed_user_w.reshape(_SUPER, 128),
                     embed_item_w.reshape(_SUPER, 128),
                     user_bias_w.reshape(-1), item_bias_w.reshape(-1))
